# Initial kernel scaffold; baseline (speedup 1.0000x reference)
import jax
import jax.numpy as jnp
from jax.experimental import pallas as pl


def kernel(x, edge_index, edge_weight, W_enc, W_b1, W_b2, W_dec, gamma, beta):
    raise NotImplementedError("write your pallas kernel here")


# speedup vs baseline: 3.3464x; 3.3464x over previous
import functools

import jax
import jax.numpy as jnp
from jax import lax
from jax.experimental import pallas as pl
from jax.experimental.pallas import tpu as pltpu
from jax.experimental.pallas import tpu_sc as plsc

N_NODES_REF = 10000
MAX_ITER = 20
TOL = 3e-06
PHANTOM_GRAD = 2

NP = 10240
HID = 256
HALF = 128
LANES = 16
N_TILES = 16
N_CORES = 2
CHUNK = 128
ROWS_PER_TILE = NP // N_TILES
UPD_CHUNK = 64
BLK = 256


def _bias_body(gi_ref, x_ref, we_ref, w1_ref, w2_ref, b_ref):
    h = jnp.dot(x_ref[...], we_ref[...], preferred_element_type=jnp.float32)
    t = jnp.maximum(jnp.dot(h, w1_ref[...], preferred_element_type=jnp.float32), 0.0)
    b = jnp.dot(t, w2_ref[...], preferred_element_type=jnp.float32)
    b_ref[...] = b * gi_ref[0, 0]


def _bias_call(xp, weT, w1T, w2T, inv_gamma):
    return pl.pallas_call(
        _bias_body,
        grid=(NP // BLK,),
        in_specs=[
            pl.BlockSpec(memory_space=pltpu.SMEM),
            pl.BlockSpec((BLK, HALF), lambda i: (i, 0)),
            pl.BlockSpec((HALF, HID), lambda i: (0, 0)),
            pl.BlockSpec((HID, HID), lambda i: (0, 0)),
            pl.BlockSpec((HID, HID), lambda i: (0, 0)),
        ],
        out_specs=pl.BlockSpec((BLK, HID), lambda i: (i, 0)),
        out_shape=jax.ShapeDtypeStruct((NP, HID), jnp.float32),
    )(inv_gamma, xp, weT, w1T, w2T)


def _dec_body(za_ref, zb_ref, wa_ref, wb_ref, o_ref):
    za = jnp.maximum(za_ref[...], 0.0)
    zb = jnp.maximum(zb_ref[...], 0.0)
    o = jnp.dot(za, wa_ref[...], preferred_element_type=jnp.float32)
    o += jnp.dot(zb, wb_ref[...], preferred_element_type=jnp.float32)
    o_ref[...] = o


def _dec_call(z_stk, waT, wbT):
    nb = NP // BLK
    return pl.pallas_call(
        _dec_body,
        grid=(nb,),
        in_specs=[
            pl.BlockSpec((BLK, HALF), lambda i: (i, 0)),
            pl.BlockSpec((BLK, HALF), lambda i, _nb=nb: (i + _nb, 0)),
            pl.BlockSpec((HALF, HALF), lambda i: (0, 0)),
            pl.BlockSpec((HALF, HALF), lambda i: (0, 0)),
        ],
        out_specs=pl.BlockSpec((BLK, HALF), lambda i: (i, 0)),
        out_shape=jax.ShapeDtypeStruct((NP, HALF), jnp.float32),
    )(z_stk, z_stk, waT, wbT)


def _make_sc_step(epad):
    e_per_tile = epad // N_TILES
    n_chunks = e_per_tile // CHUNK
    mesh = plsc.VectorSubcoreMesh(core_axis_name="c", subcore_axis_name="s")

    @functools.partial(
        pl.kernel,
        mesh=mesh,
        out_type=[
            jax.ShapeDtypeStruct((2 * NP, HALF), jnp.float32),
            jax.ShapeDtypeStruct((N_CORES * N_TILES, LANES), jnp.float32),
        ],
        scratch_types=[
            pltpu.VMEM((CHUNK,), jnp.int32),
            pltpu.VMEM((CHUNK,), jnp.int32),
            pltpu.VMEM((CHUNK,), jnp.float32),
            pltpu.VMEM((CHUNK, HALF), jnp.float32),
            pltpu.VMEM((UPD_CHUNK, HALF), jnp.float32),
            pltpu.VMEM((UPD_CHUNK, HALF), jnp.float32),
            pltpu.VMEM((3, LANES), jnp.float32),
            pltpu.VMEM((LANES,), jnp.float32),
            pltpu.VMEM_SHARED((NP, HALF), jnp.float32),
            pltpu.SemaphoreType.DMA,
        ],
    )
    def step(z_hbm, b_hbm, src_hbm, dst_hbm, w_hbm, par_hbm,
             zout_hbm, err_hbm,
             src_v, dst_v, w_v, rows_v, acc_v, zc_v, par_v, err_v, acc_sh, sem):
        c = lax.axis_index("c")
        s = lax.axis_index("s")
        row0 = s * ROWS_PER_TILE
        zbase = c * NP

        pltpu.sync_copy(par_hbm, par_v)
        pltpu.sync_copy(
            b_hbm.at[pl.ds(zbase + row0, ROWS_PER_TILE)],
            acc_sh.at[pl.ds(row0, ROWS_PER_TILE)],
        )
        plsc.subcore_barrier()

        tile_e0 = s * e_per_tile

        def chunk_body(k, carry):
            base = tile_e0 + k * CHUNK
            pltpu.sync_copy(src_hbm.at[pl.ds(base, CHUNK)], src_v)
            pltpu.sync_copy(dst_hbm.at[pl.ds(base, CHUNK)], dst_v)
            pltpu.sync_copy(w_hbm.at[pl.ds(base, CHUNK)], w_v)
            for j in range(CHUNK // LANES):
                sl = pl.ds(j * LANES, LANES)
                src_v[sl] = src_v[sl] + zbase
            pltpu.async_copy(z_hbm.at[src_v], rows_v, sem).wait()

            for g in range(CHUNK // LANES):
                wv = w_v[pl.ds(g * LANES, LANES)]
                for ee in range(LANES):
                    wb = jnp.take_along_axis(
                        wv, jnp.full((LANES,), ee, jnp.int32), axis=0,
                        mode="promise_in_bounds")
                    erow = g * LANES + ee
                    for j in range(HALF // LANES):
                        sl = pl.ds(j * LANES, LANES)
                        rows_v[erow, sl] = rows_v[erow, sl] * wb

            pltpu.sync_copy(rows_v, acc_sh.at[dst_v], add=True)
            return carry

        lax.fori_loop(0, n_chunks, chunk_body, 0)
        plsc.subcore_barrier()

        gam = par_v[0, :]
        bet = par_v[1, :]
        omb = par_v[2, :]

        def upd_body(k, err):
            r0 = row0 + k * UPD_CHUNK
            pltpu.sync_copy(acc_sh.at[pl.ds(r0, UPD_CHUNK)], acc_v)
            pltpu.sync_copy(z_hbm.at[pl.ds(zbase + r0, UPD_CHUNK)], zc_v)

            def row_body(r, e):
                for j in range(HALF // LANES):
                    sl = pl.ds(j * LANES, LANES)
                    zo = zc_v[r, sl]
                    zh = jnp.maximum(acc_v[r, sl] * gam, 0.0)
                    zn = omb * zo + bet * zh
                    acc_v[r, sl] = zn
                    e = jnp.maximum(e, jnp.abs(zn - zo))
                return e

            err = lax.fori_loop(0, UPD_CHUNK, row_body, err)
            pltpu.sync_copy(acc_v, zout_hbm.at[pl.ds(zbase + r0, UPD_CHUNK)])
            return err

        err = lax.fori_loop(0, ROWS_PER_TILE // UPD_CHUNK, upd_body,
                            jnp.zeros((LANES,), jnp.float32))
        err_v[...] = err
        wid = s * N_CORES + c
        pltpu.sync_copy(err_v, err_hbm.at[wid])

    return step


def kernel(x, edge_index, edge_weight, W_enc, W_b1, W_b2, W_dec, gamma, beta):
    n = x.shape[0]
    e = edge_weight.shape[0]
    egrp = N_TILES * CHUNK
    epad = ((e + egrp - 1) // egrp) * egrp

    xp = jnp.pad(x.astype(jnp.float32), ((0, NP - n), (0, 0)))
    src = jnp.pad(edge_index[0].astype(jnp.int32), (0, epad - e))
    dst = jnp.pad(edge_index[1].astype(jnp.int32), (0, epad - e))
    w = jnp.pad(edge_weight.astype(jnp.float32), (0, epad - e))

    gamma = gamma.astype(jnp.float32)
    beta = beta.astype(jnp.float32)
    inv_gamma = (1.0 / gamma).reshape(1, 1)
    params = jnp.stack([
        jnp.full((LANES,), gamma, jnp.float32),
        jnp.full((LANES,), beta, jnp.float32),
        jnp.full((LANES,), 1.0 - beta, jnp.float32),
    ])

    b_s = _bias_call(xp, W_enc.T, W_b1.T, W_b2.T, inv_gamma)
    b_stk = jnp.concatenate([b_s[:, :HALF], b_s[:, HALF:]], axis=0)

    step = _make_sc_step(epad)
    z0 = jnp.zeros((2 * NP, HALF), jnp.float32)

    def cond_fn(carry):
        _, i, err = carry
        return jnp.logical_and(i < MAX_ITER, err >= TOL)

    def body_fn(carry):
        z, i, _ = carry
        zn, errp = step(z, b_stk, src, dst, w, params)
        return (zn, i + 1, jnp.max(errp))

    z, _, _ = lax.while_loop(
        cond_fn, body_fn,
        (z0, jnp.asarray(0, jnp.int32), jnp.asarray(jnp.inf, jnp.float32)))

    for _ in range(PHANTOM_GRAD):
        z, _ = step(z, b_stk, src, dst, w, params)

    out = _dec_call(z, W_dec[:, :HALF].T, W_dec[:, HALF:].T)
    return out[:n]


# speedup vs baseline: 3.9705x; 1.1865x over previous
import functools

import jax
import jax.numpy as jnp
from jax import lax
from jax.experimental import pallas as pl
from jax.experimental.pallas import tpu as pltpu
from jax.experimental.pallas import tpu_sc as plsc

N_NODES_REF = 10000
MAX_ITER = 20
TOL = 3e-06
PHANTOM_GRAD = 2

NP = 10240
HID = 256
HALF = 128
LANES = 16
N_TILES = 16
N_CORES = 2
CHUNK = 128
ROWS_PER_TILE = NP // N_TILES
UPD_CHUNK = 128
BLK = 256


def _bias_body(gi_ref, x_ref, we_ref, w1_ref, w2_ref, b_ref):
    h = jnp.dot(x_ref[...], we_ref[...], preferred_element_type=jnp.float32)
    t = jnp.maximum(jnp.dot(h, w1_ref[...], preferred_element_type=jnp.float32), 0.0)
    b = jnp.dot(t, w2_ref[...], preferred_element_type=jnp.float32)
    b_ref[...] = b * gi_ref[0, 0]


def _bias_call(xp, weT, w1T, w2T, inv_gamma):
    return pl.pallas_call(
        _bias_body,
        grid=(NP // BLK,),
        in_specs=[
            pl.BlockSpec(memory_space=pltpu.SMEM),
            pl.BlockSpec((BLK, HALF), lambda i: (i, 0)),
            pl.BlockSpec((HALF, HID), lambda i: (0, 0)),
            pl.BlockSpec((HID, HID), lambda i: (0, 0)),
            pl.BlockSpec((HID, HID), lambda i: (0, 0)),
        ],
        out_specs=pl.BlockSpec((BLK, HID), lambda i: (i, 0)),
        out_shape=jax.ShapeDtypeStruct((NP, HID), jnp.float32),
    )(inv_gamma, xp, weT, w1T, w2T)


def _dec_body(za_ref, zb_ref, wa_ref, wb_ref, o_ref):
    za = jnp.maximum(za_ref[...], 0.0)
    zb = jnp.maximum(zb_ref[...], 0.0)
    o = jnp.dot(za, wa_ref[...], preferred_element_type=jnp.float32)
    o += jnp.dot(zb, wb_ref[...], preferred_element_type=jnp.float32)
    o_ref[...] = o


def _dec_call(z_stk, waT, wbT):
    nb = NP // BLK
    return pl.pallas_call(
        _dec_body,
        grid=(nb,),
        in_specs=[
            pl.BlockSpec((BLK, HALF), lambda i: (i, 0)),
            pl.BlockSpec((BLK, HALF), lambda i, _nb=nb: (i + _nb, 0)),
            pl.BlockSpec((HALF, HALF), lambda i: (0, 0)),
            pl.BlockSpec((HALF, HALF), lambda i: (0, 0)),
        ],
        out_specs=pl.BlockSpec((BLK, HALF), lambda i: (i, 0)),
        out_shape=jax.ShapeDtypeStruct((NP, HALF), jnp.float32),
    )(z_stk, z_stk, waT, wbT)


def _mult_chunk(wbuf, rows):
    for g in range(CHUNK // LANES):
        wv = wbuf[pl.ds(g * LANES, LANES)]
        for ee in range(LANES):
            wb = jnp.take_along_axis(
                wv, jnp.full((LANES,), ee, jnp.int32), axis=0,
                mode="promise_in_bounds")
            erow = g * LANES + ee
            for j in range(HALF // LANES):
                sl = pl.ds(j * LANES, LANES)
                rows[erow, sl] = rows[erow, sl] * wb


def _make_sc_step(cpt, tot):
    mesh = plsc.VectorSubcoreMesh(core_axis_name="c", subcore_axis_name="s")

    @functools.partial(
        pl.kernel,
        mesh=mesh,
        out_type=[
            jax.ShapeDtypeStruct((2 * NP, HALF), jnp.float32),
            jax.ShapeDtypeStruct((N_CORES * N_TILES, LANES), jnp.float32),
        ],
        scratch_types=[
            pltpu.VMEM((2, CHUNK), jnp.int32),
            pltpu.VMEM((2, CHUNK), jnp.int32),
            pltpu.VMEM((CHUNK,), jnp.float32),
            pltpu.VMEM((CHUNK,), jnp.float32),
            pltpu.VMEM((CHUNK, HALF), jnp.float32),
            pltpu.VMEM((CHUNK, HALF), jnp.float32),
            pltpu.VMEM((3, LANES), jnp.float32),
            pltpu.VMEM((LANES,), jnp.float32),
            pltpu.VMEM_SHARED((NP, HALF), jnp.float32),
            pltpu.SemaphoreType.DMA,
            pltpu.SemaphoreType.DMA,
            pltpu.SemaphoreType.DMA,
            pltpu.SemaphoreType.DMA,
        ],
    )
    def step(z_hbm, b_hbm, e_hbm, w_hbm, par_hbm,
             zout_hbm, err_hbm,
             eb0, eb1, wb0, wb1, rw0, rw1, par_v, err_v, acc_sh,
             es0, es1, rs0, rs1):
        ebufs = (eb0, eb1)
        wbufs = (wb0, wb1)
        rows = (rw0, rw1)
        esem = (es0, es1)
        rsem = (rs0, rs1)
        c = lax.axis_index("c")
        s = lax.axis_index("s")
        row0 = s * ROWS_PER_TILE
        zbase = c * NP

        pltpu.sync_copy(par_hbm, par_v)
        pltpu.sync_copy(
            b_hbm.at[pl.ds(zbase + row0, ROWS_PER_TILE)],
            acc_sh.at[pl.ds(row0, ROWS_PER_TILE)],
        )
        plsc.subcore_barrier()

        ebase = c * tot + s * cpt
        wbase = s * cpt

        pltpu.sync_copy(e_hbm.at[ebase], ebufs[0])
        pltpu.sync_copy(w_hbm.at[wbase], wbufs[0])
        pltpu.async_copy(z_hbm.at[ebufs[0].at[0]], rows[0], rsem[0])

        def pair_body(t, carry):
            k0 = t * 2
            for p in (0, 1):
                k = k0 + p
                pltpu.sync_copy(e_hbm.at[ebase + k + 1], ebufs[1 - p])
                pltpu.sync_copy(w_hbm.at[wbase + k + 1], wbufs[1 - p])
                pltpu.async_copy(
                    z_hbm.at[ebufs[1 - p].at[0]], rows[1 - p], rsem[1 - p])
                pltpu.make_async_copy(
                    z_hbm.at[ebufs[p].at[0]], rows[p], rsem[p]).wait()
                _mult_chunk(wbufs[p], rows[p])
                pltpu.sync_copy(rows[p], acc_sh.at[ebufs[p].at[1]], add=True)
            return carry

        lax.fori_loop(0, cpt // 2, pair_body, 0)
        pltpu.make_async_copy(z_hbm.at[ebufs[0].at[0]], rows[0], rsem[0]).wait()
        plsc.subcore_barrier()

        gam = par_v[0, :]
        bet = par_v[1, :]
        omb = par_v[2, :]

        def upd_body(k, err):
            r0 = row0 + k * UPD_CHUNK
            pltpu.sync_copy(acc_sh.at[pl.ds(r0, UPD_CHUNK)], rows[0])
            pltpu.sync_copy(z_hbm.at[pl.ds(zbase + r0, UPD_CHUNK)], rows[1])

            def row_body(r, e):
                for j in range(HALF // LANES):
                    sl = pl.ds(j * LANES, LANES)
                    zo = rows[1][r, sl]
                    zh = jnp.maximum(rows[0][r, sl] * gam, 0.0)
                    zn = omb * zo + bet * zh
                    rows[0][r, sl] = zn
                    e = jnp.maximum(e, jnp.abs(zn - zo))
                return e

            err = lax.fori_loop(0, UPD_CHUNK, row_body, err)
            pltpu.sync_copy(rows[0], zout_hbm.at[pl.ds(zbase + r0, UPD_CHUNK)])
            return err

        err = lax.fori_loop(0, ROWS_PER_TILE // UPD_CHUNK, upd_body,
                            jnp.zeros((LANES,), jnp.float32))
        err_v[...] = err
        wid = s * N_CORES + c
        pltpu.sync_copy(err_v, err_hbm.at[wid])

    return step


def kernel(x, edge_index, edge_weight, W_enc, W_b1, W_b2, W_dec, gamma, beta):
    n = x.shape[0]
    e = edge_weight.shape[0]
    egrp = N_TILES * CHUNK
    cpt = -(-e // egrp)
    cpt += cpt % 2
    tot = N_TILES * cpt + 2
    epad = tot * CHUNK

    xp = jnp.pad(x.astype(jnp.float32), ((0, NP - n), (0, 0)))
    src = jnp.pad(edge_index[0].astype(jnp.int32), (0, epad - e))
    dst = jnp.pad(edge_index[1].astype(jnp.int32), (0, epad - e))
    w = jnp.pad(edge_weight.astype(jnp.float32), (0, epad - e))

    packed = jnp.stack([
        jnp.stack([src, dst]),
        jnp.stack([src + NP, dst]),
    ])
    packed = packed.reshape(2, 2, tot, CHUNK).transpose(0, 2, 1, 3)
    packed = packed.reshape(2 * tot, 2, CHUNK)
    wchunks = w.reshape(tot, CHUNK)

    gamma = gamma.astype(jnp.float32)
    beta = beta.astype(jnp.float32)
    inv_gamma = (1.0 / gamma).reshape(1, 1)
    params = jnp.stack([
        jnp.full((LANES,), gamma, jnp.float32),
        jnp.full((LANES,), beta, jnp.float32),
        jnp.full((LANES,), 1.0 - beta, jnp.float32),
    ])

    b_s = _bias_call(xp, W_enc.T, W_b1.T, W_b2.T, inv_gamma)
    b_stk = jnp.concatenate([b_s[:, :HALF], b_s[:, HALF:]], axis=0)

    step = _make_sc_step(cpt, tot)
    z0 = jnp.zeros((2 * NP, HALF), jnp.float32)

    def cond_fn(carry):
        _, i, err = carry
        return jnp.logical_and(i < MAX_ITER, err >= TOL)

    def body_fn(carry):
        z, i, _ = carry
        zn, errp = step(z, b_stk, packed, wchunks, params)
        return (zn, i + 1, jnp.max(errp))

    z, _, _ = lax.while_loop(
        cond_fn, body_fn,
        (z0, jnp.asarray(0, jnp.int32), jnp.asarray(jnp.inf, jnp.float32)))

    for _ in range(PHANTOM_GRAD):
        z, _ = step(z, b_stk, packed, wchunks, params)

    out = _dec_call(z, W_dec[:, :HALF].T, W_dec[:, HALF:].T)
    return out[:n]


# speedup vs baseline: 4.4200x; 1.1132x over previous
import functools

import jax
import jax.numpy as jnp
from jax import lax
from jax.experimental import pallas as pl
from jax.experimental.pallas import tpu as pltpu
from jax.experimental.pallas import tpu_sc as plsc

N_NODES_REF = 10000
MAX_ITER = 20
TOL = 3e-06
PHANTOM_GRAD = 2

NP = 10240
HID = 256
HALF = 128
LANES = 16
N_TILES = 16
N_CORES = 2
CHUNK = 128
ROWS_PER_TILE = NP // N_TILES
UPD_CHUNK = 128
BLK = 256


def _bias_body(gi_ref, x_ref, we_ref, w1_ref, w2_ref, b_ref):
    h = jnp.dot(x_ref[...], we_ref[...], preferred_element_type=jnp.float32)
    t = jnp.maximum(jnp.dot(h, w1_ref[...], preferred_element_type=jnp.float32), 0.0)
    b = jnp.dot(t, w2_ref[...], preferred_element_type=jnp.float32)
    b_ref[...] = b * gi_ref[0, 0]


def _bias_call(xp, weT, w1T, w2T, inv_gamma):
    return pl.pallas_call(
        _bias_body,
        grid=(NP // BLK,),
        in_specs=[
            pl.BlockSpec(memory_space=pltpu.SMEM),
            pl.BlockSpec((BLK, HALF), lambda i: (i, 0)),
            pl.BlockSpec((HALF, HID), lambda i: (0, 0)),
            pl.BlockSpec((HID, HID), lambda i: (0, 0)),
            pl.BlockSpec((HID, HID), lambda i: (0, 0)),
        ],
        out_specs=pl.BlockSpec((BLK, HID), lambda i: (i, 0)),
        out_shape=jax.ShapeDtypeStruct((NP, HID), jnp.float32),
    )(inv_gamma, xp, weT, w1T, w2T)


def _dec_body(za_ref, zb_ref, wa_ref, wb_ref, o_ref):
    za = jnp.maximum(za_ref[...], 0.0)
    zb = jnp.maximum(zb_ref[...], 0.0)
    o = jnp.dot(za, wa_ref[...], preferred_element_type=jnp.float32)
    o += jnp.dot(zb, wb_ref[...], preferred_element_type=jnp.float32)
    o_ref[...] = o


def _dec_call(z_stk, waT, wbT):
    nb = NP // BLK
    return pl.pallas_call(
        _dec_body,
        grid=(nb,),
        in_specs=[
            pl.BlockSpec((BLK, HALF), lambda i: (i, 0)),
            pl.BlockSpec((BLK, HALF), lambda i, _nb=nb: (i + _nb, 0)),
            pl.BlockSpec((HALF, HALF), lambda i: (0, 0)),
            pl.BlockSpec((HALF, HALF), lambda i: (0, 0)),
        ],
        out_specs=pl.BlockSpec((BLK, HALF), lambda i: (i, 0)),
        out_shape=jax.ShapeDtypeStruct((NP, HALF), jnp.float32),
    )(z_stk, z_stk, waT, wbT)


def _mult_chunk(wbuf, rows):
    for g in range(CHUNK // LANES):
        wv = wbuf[pl.ds(g * LANES, LANES)]
        for ee in range(LANES):
            wb = jnp.take_along_axis(
                wv, jnp.full((LANES,), ee, jnp.int32), axis=0,
                mode="promise_in_bounds")
            erow = g * LANES + ee
            for j in range(HALF // LANES):
                sl = pl.ds(j * LANES, LANES)
                rows[erow, sl] = rows[erow, sl] * wb


def _make_sc_step(cpt, tot):
    mesh = plsc.VectorSubcoreMesh(core_axis_name="c", subcore_axis_name="s")

    @functools.partial(
        pl.kernel,
        mesh=mesh,
        out_type=[
            jax.ShapeDtypeStruct((2 * NP, HALF), jnp.float32),
            jax.ShapeDtypeStruct((N_CORES * N_TILES, LANES), jnp.float32),
        ],
        scratch_types=[
            pltpu.VMEM((2, CHUNK), jnp.int32),
            pltpu.VMEM((2, CHUNK), jnp.int32),
            pltpu.VMEM((CHUNK,), jnp.float32),
            pltpu.VMEM((CHUNK,), jnp.float32),
            pltpu.VMEM((CHUNK, HALF), jnp.float32),
            pltpu.VMEM((CHUNK, HALF), jnp.float32),
            pltpu.VMEM((3, LANES), jnp.float32),
            pltpu.VMEM((LANES,), jnp.float32),
            pltpu.VMEM_SHARED((NP, HALF), jnp.float32),
            pltpu.SemaphoreType.DMA,
            pltpu.SemaphoreType.DMA,
            pltpu.SemaphoreType.DMA,
            pltpu.SemaphoreType.DMA,
            pltpu.SemaphoreType.DMA,
            pltpu.SemaphoreType.DMA,
        ],
    )
    def step(z_hbm, b_hbm, e_hbm, w_hbm, par_hbm,
             zout_hbm, err_hbm,
             eb0, eb1, wb0, wb1, rw0, rw1, par_v, err_v, acc_sh,
             es0, es1, rs0, rs1, ss0, ss1):
        ebufs = (eb0, eb1)
        wbufs = (wb0, wb1)
        rows = (rw0, rw1)
        esem = (es0, es1)
        rsem = (rs0, rs1)
        ssem = (ss0, ss1)
        c = lax.axis_index("c")
        s = lax.axis_index("s")
        row0 = s * ROWS_PER_TILE
        zbase = c * NP

        pltpu.sync_copy(par_hbm, par_v)
        pltpu.sync_copy(
            b_hbm.at[pl.ds(zbase + row0, ROWS_PER_TILE)],
            acc_sh.at[pl.ds(row0, ROWS_PER_TILE)],
        )
        plsc.subcore_barrier()

        ebase = c * tot + s * cpt
        wbase = s * cpt

        pltpu.sync_copy(e_hbm.at[ebase], ebufs[0])
        pltpu.sync_copy(w_hbm.at[wbase], wbufs[0])
        pltpu.async_copy(z_hbm.at[ebufs[0].at[0]], rows[0], rsem[0])

        def pair_body(t, carry):
            k0 = t * 2
            for p in (0, 1):
                k = k0 + p
                pltpu.sync_copy(e_hbm.at[ebase + k + 1], ebufs[1 - p])
                pltpu.sync_copy(w_hbm.at[wbase + k + 1], wbufs[1 - p])
                if p == 1:
                    pltpu.make_async_copy(
                        rows[0], acc_sh.at[ebufs[0].at[1]], ssem[0]).wait()
                else:
                    @pl.when(t > 0)
                    def _():
                        pltpu.make_async_copy(
                            rows[1], acc_sh.at[ebufs[1].at[1]], ssem[1]).wait()
                pltpu.async_copy(
                    z_hbm.at[ebufs[1 - p].at[0]], rows[1 - p], rsem[1 - p])
                pltpu.make_async_copy(
                    z_hbm.at[ebufs[p].at[0]], rows[p], rsem[p]).wait()
                _mult_chunk(wbufs[p], rows[p])
                pltpu.async_copy(
                    rows[p], acc_sh.at[ebufs[p].at[1]], ssem[p], add=True)
            return carry

        lax.fori_loop(0, cpt // 2, pair_body, 0)
        pltpu.make_async_copy(z_hbm.at[ebufs[0].at[0]], rows[0], rsem[0]).wait()
        pltpu.make_async_copy(rows[1], acc_sh.at[ebufs[1].at[1]], ssem[1]).wait()
        plsc.subcore_barrier()

        gam = par_v[0, :]
        bet = par_v[1, :]
        omb = par_v[2, :]

        def upd_body(k, err):
            r0 = row0 + k * UPD_CHUNK
            pltpu.sync_copy(acc_sh.at[pl.ds(r0, UPD_CHUNK)], rows[0])
            pltpu.sync_copy(z_hbm.at[pl.ds(zbase + r0, UPD_CHUNK)], rows[1])

            def row_body(r, e):
                for j in range(HALF // LANES):
                    sl = pl.ds(j * LANES, LANES)
                    zo = rows[1][r, sl]
                    zh = jnp.maximum(rows[0][r, sl] * gam, 0.0)
                    zn = omb * zo + bet * zh
                    rows[0][r, sl] = zn
                    e = jnp.maximum(e, jnp.abs(zn - zo))
                return e

            err = lax.fori_loop(0, UPD_CHUNK, row_body, err)
            pltpu.sync_copy(rows[0], zout_hbm.at[pl.ds(zbase + r0, UPD_CHUNK)])
            return err

        err = lax.fori_loop(0, ROWS_PER_TILE // UPD_CHUNK, upd_body,
                            jnp.zeros((LANES,), jnp.float32))
        err_v[...] = err
        wid = s * N_CORES + c
        pltpu.sync_copy(err_v, err_hbm.at[wid])

    return step


def kernel(x, edge_index, edge_weight, W_enc, W_b1, W_b2, W_dec, gamma, beta):
    n = x.shape[0]
    e = edge_weight.shape[0]
    egrp = N_TILES * CHUNK
    cpt = -(-e // egrp)
    cpt += cpt % 2
    tot = N_TILES * cpt + 2
    epad = tot * CHUNK

    xp = jnp.pad(x.astype(jnp.float32), ((0, NP - n), (0, 0)))
    src = jnp.pad(edge_index[0].astype(jnp.int32), (0, epad - e))
    dst = jnp.pad(edge_index[1].astype(jnp.int32), (0, epad - e))
    w = jnp.pad(edge_weight.astype(jnp.float32), (0, epad - e))

    packed = jnp.stack([
        jnp.stack([src, dst]),
        jnp.stack([src + NP, dst]),
    ])
    packed = packed.reshape(2, 2, tot, CHUNK).transpose(0, 2, 1, 3)
    packed = packed.reshape(2 * tot, 2, CHUNK)
    wchunks = w.reshape(tot, CHUNK)

    gamma = gamma.astype(jnp.float32)
    beta = beta.astype(jnp.float32)
    inv_gamma = (1.0 / gamma).reshape(1, 1)
    params = jnp.stack([
        jnp.full((LANES,), gamma, jnp.float32),
        jnp.full((LANES,), beta, jnp.float32),
        jnp.full((LANES,), 1.0 - beta, jnp.float32),
    ])

    b_s = _bias_call(xp, W_enc.T, W_b1.T, W_b2.T, inv_gamma)
    b_stk = jnp.concatenate([b_s[:, :HALF], b_s[:, HALF:]], axis=0)

    step = _make_sc_step(cpt, tot)
    z0 = jnp.zeros((2 * NP, HALF), jnp.float32)

    def cond_fn(carry):
        _, i, err = carry
        return jnp.logical_and(i < MAX_ITER, err >= TOL)

    def body_fn(carry):
        z, i, _ = carry
        zn, errp = step(z, b_stk, packed, wchunks, params)
        return (zn, i + 1, jnp.max(errp))

    z, _, _ = lax.while_loop(
        cond_fn, body_fn,
        (z0, jnp.asarray(0, jnp.int32), jnp.asarray(jnp.inf, jnp.float32)))

    for _ in range(PHANTOM_GRAD):
        z, _ = step(z, b_stk, packed, wchunks, params)

    out = _dec_call(z, W_dec[:, :HALF].T, W_dec[:, HALF:].T)
    return out[:n]


# speedup vs baseline: 8.3100x; 1.8801x over previous
import functools

import jax
import jax.numpy as jnp
from jax import lax
from jax.experimental import pallas as pl
from jax.experimental.pallas import tpu as pltpu
from jax.experimental.pallas import tpu_sc as plsc

N_NODES_REF = 10000
MAX_ITER = 20
TOL = 3e-06
PHANTOM_GRAD = 2

NP = 10240
HID = 256
HALF = 128
LANES = 16
N_TILES = 16
N_CORES = 2
CHUNK = 128
ROWS_PER_TILE = NP // N_TILES
UPD_CHUNK = 128
BLK = 256


def _bias_body(gi_ref, x_ref, we_ref, w1_ref, w2_ref, b_ref):
    h = jnp.dot(x_ref[...], we_ref[...], preferred_element_type=jnp.float32)
    t = jnp.maximum(jnp.dot(h, w1_ref[...], preferred_element_type=jnp.float32), 0.0)
    b = jnp.dot(t, w2_ref[...], preferred_element_type=jnp.float32)
    b_ref[...] = b * gi_ref[0, 0]


def _bias_call(xp, weT, w1T, w2T, inv_gamma):
    return pl.pallas_call(
        _bias_body,
        grid=(NP // BLK,),
        in_specs=[
            pl.BlockSpec(memory_space=pltpu.SMEM),
            pl.BlockSpec((BLK, HALF), lambda i: (i, 0)),
            pl.BlockSpec((HALF, HID), lambda i: (0, 0)),
            pl.BlockSpec((HID, HID), lambda i: (0, 0)),
            pl.BlockSpec((HID, HID), lambda i: (0, 0)),
        ],
        out_specs=pl.BlockSpec((BLK, HID), lambda i: (i, 0)),
        out_shape=jax.ShapeDtypeStruct((NP, HID), jnp.float32),
    )(inv_gamma, xp, weT, w1T, w2T)


def _dec_body(za_ref, zb_ref, wa_ref, wb_ref, o_ref):
    za = jnp.maximum(za_ref[...], 0.0)
    zb = jnp.maximum(zb_ref[...], 0.0)
    o = jnp.dot(za, wa_ref[...], preferred_element_type=jnp.float32)
    o += jnp.dot(zb, wb_ref[...], preferred_element_type=jnp.float32)
    o_ref[...] = o


def _dec_call(z_stk, waT, wbT):
    nb = NP // BLK
    return pl.pallas_call(
        _dec_body,
        grid=(nb,),
        in_specs=[
            pl.BlockSpec((BLK, HALF), lambda i: (i, 0)),
            pl.BlockSpec((BLK, HALF), lambda i, _nb=nb: (i + _nb, 0)),
            pl.BlockSpec((HALF, HALF), lambda i: (0, 0)),
            pl.BlockSpec((HALF, HALF), lambda i: (0, 0)),
        ],
        out_specs=pl.BlockSpec((BLK, HALF), lambda i: (i, 0)),
        out_shape=jax.ShapeDtypeStruct((NP, HALF), jnp.float32),
    )(z_stk, z_stk, waT, wbT)


def _mult_chunk(wbuf, j, rows):
    for g in range(CHUNK // LANES):
        wv = wbuf[j, pl.ds(g * LANES, LANES)]
        for ee in range(LANES):
            wb = jnp.take_along_axis(
                wv, jnp.full((LANES,), ee, jnp.int32), axis=0,
                mode="promise_in_bounds")
            erow = g * LANES + ee
            for j in range(HALF // LANES):
                sl = pl.ds(j * LANES, LANES)
                rows[erow, sl] = rows[erow, sl] * wb


def _make_sc_step(cpt, tot):
    mesh = plsc.VectorSubcoreMesh(core_axis_name="c", subcore_axis_name="s")

    @functools.partial(
        pl.kernel,
        mesh=mesh,
        out_type=[
            jax.ShapeDtypeStruct((2 * NP, HALF), jnp.float32),
            jax.ShapeDtypeStruct((N_CORES * N_TILES, LANES), jnp.float32),
        ],
        scratch_types=[
            pltpu.VMEM((2, 2, CHUNK), jnp.int32),
            pltpu.VMEM((2, 2, CHUNK), jnp.int32),
            pltpu.VMEM((2, CHUNK), jnp.float32),
            pltpu.VMEM((2, CHUNK), jnp.float32),
            pltpu.VMEM((CHUNK, HALF), jnp.float32),
            pltpu.VMEM((CHUNK, HALF), jnp.float32),
            pltpu.VMEM((3, LANES), jnp.float32),
            pltpu.VMEM((LANES,), jnp.float32),
            pltpu.VMEM_SHARED((NP, HALF), jnp.float32),
            pltpu.SemaphoreType.DMA,
            pltpu.SemaphoreType.DMA,
            pltpu.SemaphoreType.DMA,
            pltpu.SemaphoreType.DMA,
            pltpu.SemaphoreType.DMA,
            pltpu.SemaphoreType.DMA,
        ],
    )
    def step(z_hbm, b_hbm, e_hbm, w_hbm, par_hbm,
             zout_hbm, err_hbm,
             eb0, eb1, wb0, wb1, rw0, rw1, par_v, err_v, acc_sh,
             es0, es1, rs0, rs1, ss0, ss1):
        ebufs = (eb0, eb1)
        wbufs = (wb0, wb1)
        rows = (rw0, rw1)
        esem = (es0, es1)
        rsem = (rs0, rs1)
        ssem = (ss0, ss1)
        c = lax.axis_index("c")
        s = lax.axis_index("s")
        row0 = s * ROWS_PER_TILE
        zbase = c * NP

        pltpu.sync_copy(par_hbm, par_v)
        pltpu.sync_copy(
            b_hbm.at[pl.ds(zbase + row0, ROWS_PER_TILE)],
            acc_sh.at[pl.ds(row0, ROWS_PER_TILE)],
        )
        plsc.subcore_barrier()

        ebase = c * tot + s * cpt
        wbase = s * cpt

        pltpu.sync_copy(e_hbm.at[pl.ds(ebase, 2)], ebufs[0])
        pltpu.sync_copy(w_hbm.at[pl.ds(wbase, 2)], wbufs[0])
        pltpu.async_copy(z_hbm.at[ebufs[0].at[0, 0]], rows[0], rsem[0])

        def quad_body(q, carry):
            for p01 in (0, 1):
                u = q * 2 + p01
                if p01 == 1:
                    pltpu.make_async_copy(
                        rows[1], acc_sh.at[ebufs[p01].at[1, 1]], ssem[1]).wait()
                else:
                    @pl.when(u > 0)
                    def _():
                        pltpu.make_async_copy(
                            rows[1], acc_sh.at[ebufs[p01].at[1, 1]],
                            ssem[1]).wait()
                pltpu.async_copy(
                    z_hbm.at[ebufs[p01].at[1, 0]], rows[1], rsem[1])
                pltpu.make_async_copy(
                    z_hbm.at[ebufs[p01].at[0, 0]], rows[0], rsem[0]).wait()
                _mult_chunk(wbufs[p01], 0, rows[0])
                pltpu.async_copy(
                    rows[0], acc_sh.at[ebufs[p01].at[0, 1]], ssem[0], add=True)
                pltpu.sync_copy(
                    e_hbm.at[pl.ds(ebase + (u + 1) * 2, 2)], ebufs[1 - p01])
                pltpu.sync_copy(
                    w_hbm.at[pl.ds(wbase + (u + 1) * 2, 2)], wbufs[1 - p01])
                pltpu.make_async_copy(
                    rows[0], acc_sh.at[ebufs[p01].at[0, 1]], ssem[0]).wait()
                pltpu.async_copy(
                    z_hbm.at[ebufs[1 - p01].at[0, 0]], rows[0], rsem[0])
                pltpu.make_async_copy(
                    z_hbm.at[ebufs[p01].at[1, 0]], rows[1], rsem[1]).wait()
                _mult_chunk(wbufs[p01], 1, rows[1])
                pltpu.async_copy(
                    rows[1], acc_sh.at[ebufs[p01].at[1, 1]], ssem[1], add=True)
            return carry

        lax.fori_loop(0, cpt // 4, quad_body, 0)
        pltpu.make_async_copy(z_hbm.at[ebufs[0].at[0, 0]], rows[0], rsem[0]).wait()
        pltpu.make_async_copy(rows[1], acc_sh.at[ebufs[1].at[1, 1]], ssem[1]).wait()
        plsc.subcore_barrier()

        gam = par_v[0, :]
        bet = par_v[1, :]
        omb = par_v[2, :]

        def upd_body(k, err):
            r0 = row0 + k * UPD_CHUNK
            pltpu.sync_copy(acc_sh.at[pl.ds(r0, UPD_CHUNK)], rows[0])
            pltpu.sync_copy(z_hbm.at[pl.ds(zbase + r0, UPD_CHUNK)], rows[1])

            def row_body(r, e):
                for j in range(HALF // LANES):
                    sl = pl.ds(j * LANES, LANES)
                    zo = rows[1][r, sl]
                    zh = jnp.maximum(rows[0][r, sl] * gam, 0.0)
                    zn = omb * zo + bet * zh
                    rows[0][r, sl] = zn
                    e = jnp.maximum(e, jnp.abs(zn - zo))
                return e

            err = lax.fori_loop(0, UPD_CHUNK, row_body, err)
            pltpu.sync_copy(rows[0], zout_hbm.at[pl.ds(zbase + r0, UPD_CHUNK)])
            return err

        err = lax.fori_loop(0, ROWS_PER_TILE // UPD_CHUNK, upd_body,
                            jnp.zeros((LANES,), jnp.float32))
        err_v[...] = err
        wid = s * N_CORES + c
        pltpu.sync_copy(err_v, err_hbm.at[wid])

    return step


def kernel(x, edge_index, edge_weight, W_enc, W_b1, W_b2, W_dec, gamma, beta):
    n = x.shape[0]
    e = edge_weight.shape[0]
    egrp = N_TILES * CHUNK
    cpt = -(-e // egrp)
    cpt = -(-cpt // 4) * 4
    tot = N_TILES * cpt + 2
    epad = tot * CHUNK

    xp = jnp.pad(x.astype(jnp.float32), ((0, NP - n), (0, 0)))
    src = jnp.pad(edge_index[0].astype(jnp.int32), (0, epad - e))
    dst = jnp.pad(edge_index[1].astype(jnp.int32), (0, epad - e))
    w = jnp.pad(edge_weight.astype(jnp.float32), (0, epad - e))

    packed = jnp.stack([
        jnp.stack([src, dst]),
        jnp.stack([src + NP, dst]),
    ])
    packed = packed.reshape(2, 2, tot, CHUNK).transpose(0, 2, 1, 3)
    packed = packed.reshape(2 * tot, 2, CHUNK)
    wchunks = w.reshape(tot, CHUNK)

    gamma = gamma.astype(jnp.float32)
    beta = beta.astype(jnp.float32)
    inv_gamma = (1.0 / gamma).reshape(1, 1)
    params = jnp.stack([
        jnp.full((LANES,), gamma, jnp.float32),
        jnp.full((LANES,), beta, jnp.float32),
        jnp.full((LANES,), 1.0 - beta, jnp.float32),
    ])

    b_s = _bias_call(xp, W_enc.T, W_b1.T, W_b2.T, inv_gamma)
    b_stk = jnp.concatenate([b_s[:, :HALF], b_s[:, HALF:]], axis=0)

    step = _make_sc_step(cpt, tot)
    z0 = jnp.zeros((2 * NP, HALF), jnp.float32)

    def cond_fn(carry):
        _, i, err = carry
        return jnp.logical_and(i < MAX_ITER, err >= TOL)

    def body_fn(carry):
        z, i, _ = carry
        zn, errp = step(z, b_stk, packed, wchunks, params)
        return (zn, i + 1, jnp.max(errp))

    z, _, _ = lax.while_loop(
        cond_fn, body_fn,
        (z0, jnp.asarray(0, jnp.int32), jnp.asarray(jnp.inf, jnp.float32)))

    for _ in range(PHANTOM_GRAD):
        z, _ = step(z, b_stk, packed, wchunks, params)

    out = _dec_call(z, W_dec[:, :HALF].T, W_dec[:, HALF:].T)
    return out[:n]
